# baseline (speedup 1.0000x reference)
import functools

import jax
import jax.numpy as jnp
from jax import lax
from jax.experimental import pallas as pl
from jax.experimental.pallas import tpu as pltpu
from jax.experimental.pallas import tpu_sc as plsc

B = 16384
NBR = 50
D = 16
LANES = 128
NW = 32
BPW = B // NW
CB = 64
NCHUNK = BPW // CB
IDX_ROWS = CB * NBR // LANES
UROWS = BPW // LANES


def _sc_body(user_h, item_h, nbr_h, uembs_h, iembs_h,
             uout_h, iout_h, nsum_h,
             uidx_v, urows_v, nidx_v, nrows_v, nsum_v, sem):
    wid = lax.axis_index("s") * 2 + lax.axis_index("c")
    base = wid * BPW

    pltpu.sync_copy(user_h.at[pl.ds(base, BPW)], uidx_v)
    cps = [pltpu.async_copy(uembs_h.at[uidx_v.at[pl.ds(k * LANES, LANES)]],
                            urows_v.at[pl.ds(k * LANES, LANES)], sem)
           for k in range(UROWS)]
    for cp in cps:
        cp.wait()
    pltpu.sync_copy(urows_v, uout_h.at[pl.ds(base, BPW)])

    pltpu.sync_copy(item_h.at[pl.ds(base, BPW)], uidx_v)
    cps = [pltpu.async_copy(iembs_h.at[uidx_v.at[pl.ds(k * LANES, LANES)]],
                            urows_v.at[pl.ds(k * LANES, LANES)], sem)
           for k in range(UROWS)]
    for cp in cps:
        cp.wait()
    pltpu.sync_copy(urows_v, iout_h.at[pl.ds(base, BPW)])

    def chunk_body(c, carry):
        pltpu.sync_copy(
            nbr_h.at[pl.ds(base * NBR + c * CB * NBR, CB * NBR)],
            nidx_v)
        gcps = [pltpu.async_copy(uembs_h.at[nidx_v.at[pl.ds(k * LANES, LANES)]],
                                 nrows_v.at[pl.ds(k * LANES, LANES)], sem)
                for k in range(IDX_ROWS)]
        for cp in gcps:
            cp.wait()

        def rbody(b, rc):
            o = b * NBR
            accs = [nrows_v[o + j, :] for j in range(4)]
            for j in range(4, NBR):
                accs[j % 4] = accs[j % 4] + nrows_v[o + j, :]
            nsum_v[b, :] = (accs[0] + accs[1]) + (accs[2] + accs[3])
            return rc

        lax.fori_loop(0, CB, rbody, 0)
        pltpu.sync_copy(nsum_v, nsum_h.at[pl.ds(base + c * CB, CB)])
        return carry

    lax.fori_loop(0, NCHUNK, chunk_body, 0)


_sc_call = functools.partial(
    pl.kernel,
    mesh=plsc.VectorSubcoreMesh(core_axis_name="c", subcore_axis_name="s"),
    compiler_params=pltpu.CompilerParams(use_tc_tiling_on_sc=False),
    out_type=(
        jax.ShapeDtypeStruct((B, D), jnp.float32),
        jax.ShapeDtypeStruct((B, D), jnp.float32),
        jax.ShapeDtypeStruct((B, D), jnp.float32),
    ),
    scratch_types=(
        pltpu.VMEM((BPW,), jnp.int32),
        pltpu.VMEM((BPW, D), jnp.float32),
        pltpu.VMEM((CB * NBR,), jnp.int32),
        pltpu.VMEM((CB * NBR, D), jnp.float32),
        pltpu.VMEM((CB, D), jnp.float32),
        pltpu.SemaphoreType.DMA,
    ),
)(_sc_body)


TB = 2048


def _tc_body(uemb_r, iemb_r, nsum_r, nbr_r, logit_r, nbremb_r):
    ue = uemb_r[...]
    ie = iemb_r[...]
    logit_r[...] = jnp.sum(ue * ie, axis=-1)
    cnt = jnp.sum((nbr_r[...] == 0).astype(jnp.float32), axis=-1)
    ln = jnp.float32(NBR) - cnt
    nbremb_r[...] = nsum_r[...] / ln[:, None]


def _tc_call(uemb, iemb, nsum, nbr):
    return pl.pallas_call(
        _tc_body,
        grid=(B // TB,),
        in_specs=[
            pl.BlockSpec((TB, D), lambda i: (i, 0)),
            pl.BlockSpec((TB, D), lambda i: (i, 0)),
            pl.BlockSpec((TB, D), lambda i: (i, 0)),
            pl.BlockSpec((TB, NBR), lambda i: (i, 0)),
        ],
        out_specs=[
            pl.BlockSpec((TB,), lambda i: (i,)),
            pl.BlockSpec((TB, D), lambda i: (i, 0)),
        ],
        out_shape=[
            jax.ShapeDtypeStruct((B,), jnp.float32),
            jax.ShapeDtypeStruct((B, D), jnp.float32),
        ],
    )(uemb, iemb, nsum, nbr)


def kernel(user, u_ir, nbr, item, rate, user_embs, item_embs):
    nbr_flat = nbr.reshape(B * NBR)
    uemb, iemb, nsum = _sc_call(user, item, nbr_flat, user_embs, item_embs)
    logits, nbremb = _tc_call(uemb, iemb, nsum, nbr)
    return (logits, uemb, nbremb, iemb)
